# baseline (speedup 1.0000x reference)
import functools

import jax
import jax.numpy as jnp
from jax import lax
from jax.experimental import pallas as pl
from jax.experimental.pallas import tpu as pltpu
from jax.experimental.pallas import tpu_sc as plsc

EMBED = 64
BATCH = 16384
LANES = 16

_info = plsc.get_sparse_core_info()
NC = _info.num_cores
NS = _info.num_subcores
NW = NC * NS
B_PER_W = BATCH // NW
CHUNK = 128
NCHUNK = B_PER_W // CHUNK
GROUP = 16
NGROUP = B_PER_W // GROUP
NV = EMBED // LANES


def _take16(x, idx):
    return lax.gather(
        x,
        idx[:, None],
        dimension_numbers=lax.GatherDimensionNumbers(
            offset_dims=(), collapsed_slice_dims=(0,), start_index_map=(0,)
        ),
        slice_sizes=(1,),
        mode=lax.GatherScatterMode.PROMISE_IN_BOUNDS,
    )


def _rsqrt(x):
    i = lax.bitcast_convert_type(x, jnp.int32)
    i = jnp.int32(0x5F3759DF) - (i >> 1)
    y = lax.bitcast_convert_type(i, jnp.float32)
    for _ in range(3):
        y = y * (1.5 - 0.5 * x * y * y)
    return y


@functools.partial(
    pl.kernel,
    mesh=plsc.VectorSubcoreMesh(core_axis_name="c", subcore_axis_name="s"),
    out_type=jax.ShapeDtypeStruct((BATCH, EMBED), jnp.float32),
    scratch_types=[
        pltpu.VMEM((B_PER_W,), jnp.int32),
        pltpu.VMEM((NCHUNK, CHUNK), jnp.int32),
        pltpu.VMEM((2, CHUNK, 2 * EMBED), jnp.float32),
        pltpu.VMEM((B_PER_W, EMBED), jnp.float32),
        pltpu.SemaphoreType.DMA,
        pltpu.SemaphoreType.DMA,
    ],
)
def _embed_norm(table_hbm, idx_hbm, out_hbm, idx_v, pidx_v, pairs_v, outb_v, sem0, sem1):
    wid = lax.axis_index("s") * NC + lax.axis_index("c")
    base = wid * B_PER_W
    sems = (sem0, sem1)

    pltpu.sync_copy(idx_hbm.at[wid], idx_v)

    def _shift(g, carry):
        v = idx_v[pl.ds(g * GROUP, GROUP)]
        c = g // (CHUNK // GROUP)
        pidx_v[c, pl.ds((g % (CHUNK // GROUP)) * GROUP, GROUP)] = v >> 1
        return carry

    lax.fori_loop(0, NGROUP, _shift, 0)

    def _fire(c):
        return pltpu.async_copy(
            table_hbm.at[pidx_v.at[c]], pairs_v.at[c % 2], sems[c % 2]
        )

    lanes = lax.iota(jnp.int32, LANES)
    gpc = CHUNK // GROUP

    cps = {0: _fire(0), 1: _fire(1)}
    for c in range(NCHUNK):
        cps[c].wait()

        def _group(g, carry, c=c):
            rb = g * GROUP
            vidx = idx_v[pl.ds(c * CHUNK + rb, GROUP)]
            for j in range(GROUP):
                rloc = rb + j
                o64 = (vidx[j] & 1) * EMBED
                vs = [
                    pairs_v[c % 2, rloc, pl.ds(o64 + LANES * k, LANES)]
                    for k in range(NV)
                ]
                ssq = vs[0] * vs[0]
                for k in range(1, NV):
                    ssq = ssq + vs[k] * vs[k]
                for sh in (8, 4, 2, 1):
                    ssq = ssq + _take16(ssq, lanes ^ sh)
                y = _rsqrt(ssq + 1e-12)
                for k in range(NV):
                    outb_v[c * CHUNK + rloc, pl.ds(LANES * k, LANES)] = vs[k] * y
            return carry

        lax.fori_loop(0, gpc, _group, 0)
        if c + 2 < NCHUNK:
            cps[c + 2] = _fire(c + 2)

    pltpu.sync_copy(outb_v, out_hbm.at[pl.ds(base, B_PER_W)])


def kernel(indices, table):
    idx = indices.astype(jnp.int32).reshape(NW, B_PER_W)
    table2 = table.reshape(table.shape[0] // 2, 2 * EMBED)
    return _embed_norm(table2, idx)


# speedup vs baseline: 2.3944x; 2.3944x over previous
import functools

import jax
import jax.numpy as jnp
from jax import lax
from jax.experimental import pallas as pl
from jax.experimental.pallas import tpu as pltpu
from jax.experimental.pallas import tpu_sc as plsc

VOCAB = 1_000_000
EMBED = 64
BATCH = 16384
LANES = 16

_info = plsc.get_sparse_core_info()
NC = _info.num_cores
NS = _info.num_subcores
NW = NC * NS
BLK = 512
NBF = VOCAB // BLK
BPW = NBF // NW
SPAN = BPW * BLK
TAILS = NBF * BLK
TAILW = VOCAB - TAILS
IDXCH = 2048
SELCAP = 640
NV = EMBED // LANES


def _take16(x, idx):
    return lax.gather(
        x,
        idx[:, None],
        dimension_numbers=lax.GatherDimensionNumbers(
            offset_dims=(), collapsed_slice_dims=(0,), start_index_map=(0,)
        ),
        slice_sizes=(1,),
        mode=lax.GatherScatterMode.PROMISE_IN_BOUNDS,
    )


def _rsqrt(x):
    i = lax.bitcast_convert_type(x, jnp.int32)
    i = jnp.int32(0x5F3759DF) - (i >> 1)
    y = lax.bitcast_convert_type(i, jnp.float32)
    for _ in range(3):
        y = y * (1.5 - 0.5 * x * y * y)
    return y


@functools.partial(
    pl.kernel,
    mesh=plsc.VectorSubcoreMesh(core_axis_name="c", subcore_axis_name="s"),
    out_type=jax.ShapeDtypeStruct((BATCH * EMBED,), jnp.float32),
    scratch_types=[
        pltpu.VMEM((IDXCH,), jnp.int32),
        pltpu.VMEM((EMBED, BLK), jnp.float32),
        pltpu.VMEM((EMBED, TAILW), jnp.float32),
        pltpu.VMEM((SELCAP,), jnp.int32),
        pltpu.VMEM((SELCAP,), jnp.int32),
        pltpu.VMEM((SELCAP, EMBED), jnp.float32),
        pltpu.SemaphoreType.DMA,
    ],
    compiler_params=pltpu.CompilerParams(needs_layout_passes=False),
)
def _embed_norm(
    table_hbm, idx_hbm, out_hbm, idx_v, blk_v, tail_v, sel_i, sel_b, colbuf, sem_out
):
    wid = lax.axis_index("s") * NC + lax.axis_index("c")
    lo = wid * SPAN
    hi = jnp.where(wid == NW - 1, jnp.int32(VOCAB), lo + SPAN)
    lanes = lax.iota(jnp.int32, LANES)

    def _chunk(ch, cnt):
        pltpu.sync_copy(idx_hbm.at[pl.ds(ch * IDXCH, IDXCH)], idx_v)

        def _sel(v, cnt):
            x = idx_v[pl.ds(v * LANES, LANES)]
            b = lanes + (ch * IDXCH + v * LANES)
            m = (x >= lo) & (x < hi)
            pos = cnt + plsc.cumsum(m.astype(jnp.int32)) - 1
            mm = m & (pos < SELCAP)
            plsc.store_scatter(sel_i, [pos], x, mask=mm)
            plsc.store_scatter(sel_b, [pos], b, mask=mm)
            return cnt + plsc.all_reduce_population_count(m)[0]

        return lax.fori_loop(0, IDXCH // LANES, _sel, cnt)

    nsel = lax.fori_loop(0, BATCH // IDXCH, _chunk, jnp.int32(0))
    ub_sel = (nsel + LANES - 1) // LANES

    def _match(buf, start, width, s_cnt):
        def _scan(u, s_cnt):
            sv = sel_i[pl.ds(u * LANES, LANES)]
            sb = sel_b[pl.ds(u * LANES, LANES)]
            c = sv - start
            valid = (lanes + u * LANES) < nsel
            m = (c >= 0) & (c < width) & valid
            n = plsc.all_reduce_population_count(m)[0]

            def _body(carry):
                mi, sc, n = carry
                m = mi != 0
                l = plsc.all_reduce_ffs(m)[0]
                lsp = jnp.full((LANES,), l, dtype=jnp.int32)
                ci = _take16(c, lsp)[0]
                bb = _take16(sb, lsp)[0]
                civ = jnp.full((LANES,), ci, dtype=jnp.int32)
                vs = [
                    plsc.load_gather(buf, [lanes + LANES * k, civ])
                    for k in range(NV)
                ]
                ssq = vs[0] * vs[0]
                for k in range(1, NV):
                    ssq = ssq + vs[k] * vs[k]
                for sh in (8, 4, 2, 1):
                    ssq = ssq + _take16(ssq, lanes ^ sh)
                y = _rsqrt(ssq + 1e-12)
                for k in range(NV):
                    colbuf[sc, pl.ds(LANES * k, LANES)] = vs[k] * y
                pltpu.async_copy(
                    colbuf.at[sc], out_hbm.at[pl.ds(bb * EMBED, EMBED)], sem_out
                )
                return (jnp.where(lanes == l, jnp.int32(0), mi), sc + 1, n - 1)

            out = lax.while_loop(
                lambda cr: cr[2] > 0, _body, (m.astype(jnp.int32), s_cnt, n)
            )
            return out[1]

        return lax.fori_loop(0, ub_sel, _scan, s_cnt)

    def _blk(kk, s_cnt):
        def _do(s_cnt):
            start = lo + kk * BLK
            pltpu.sync_copy(table_hbm.at[:, pl.ds(start, BLK)], blk_v)
            return _match(blk_v, start, jnp.int32(BLK), s_cnt)

        active = (kk < BPW) | (wid == NW - 1)
        return lax.cond(active, _do, lambda sc: sc, s_cnt)

    s_cnt = lax.fori_loop(0, BPW + 1, _blk, jnp.int32(0))

    def _tail(s_cnt):
        pltpu.sync_copy(table_hbm.at[:, pl.ds(TAILS, TAILW)], tail_v)
        return _match(tail_v, jnp.int32(TAILS), jnp.int32(TAILW), s_cnt)

    s_cnt = lax.cond(wid == NW - 1, _tail, lambda sc: sc, s_cnt)

    def _drain(c, carry):
        pltpu.make_async_copy(
            out_hbm.at[pl.ds(0, EMBED)], colbuf.at[0], sem_out
        ).wait()
        return carry

    lax.fori_loop(0, s_cnt, _drain, 0)


def kernel(indices, table):
    idx = indices.astype(jnp.int32)
    res = _embed_norm(table.T, idx)
    return res.reshape(BATCH, EMBED)


# speedup vs baseline: 2.9391x; 1.2275x over previous
import functools

import jax
import jax.numpy as jnp
from jax import lax
from jax.experimental import pallas as pl
from jax.experimental.pallas import tpu as pltpu
from jax.experimental.pallas import tpu_sc as plsc

VOCAB = 1_000_000
EMBED = 64
BATCH = 16384
LANES = 16

_info = plsc.get_sparse_core_info()
NC = _info.num_cores
NS = _info.num_subcores
NW = NC * NS
BLK = 256
NBF = VOCAB // BLK
BPW = NBF // NW
NEXTRA = NBF - BPW * NW
TAILS = NBF * BLK
TAILW = VOCAB - TAILS
IDXCH = 2048
SELCAP = 640
NV = EMBED // LANES


def _take16(x, idx):
    return lax.gather(
        x,
        idx[:, None],
        dimension_numbers=lax.GatherDimensionNumbers(
            offset_dims=(), collapsed_slice_dims=(0,), start_index_map=(0,)
        ),
        slice_sizes=(1,),
        mode=lax.GatherScatterMode.PROMISE_IN_BOUNDS,
    )


def _rsqrt(x):
    i = lax.bitcast_convert_type(x, jnp.int32)
    i = jnp.int32(0x5F3759DF) - (i >> 1)
    y = lax.bitcast_convert_type(i, jnp.float32)
    for _ in range(3):
        y = y * (1.5 - 0.5 * x * y * y)
    return y


@functools.partial(
    pl.kernel,
    mesh=plsc.VectorSubcoreMesh(core_axis_name="c", subcore_axis_name="s"),
    out_type=jax.ShapeDtypeStruct((BATCH * EMBED,), jnp.float32),
    scratch_types=[
        pltpu.VMEM((IDXCH,), jnp.int32),
        pltpu.VMEM((2, EMBED, BLK), jnp.float32),
        pltpu.VMEM((EMBED, TAILW), jnp.float32),
        pltpu.VMEM((SELCAP,), jnp.int32),
        pltpu.VMEM((SELCAP,), jnp.int32),
        pltpu.VMEM((SELCAP, EMBED), jnp.float32),
        pltpu.SemaphoreType.DMA,
        pltpu.SemaphoreType.DMA,
        pltpu.SemaphoreType.DMA,
    ],
    compiler_params=pltpu.CompilerParams(needs_layout_passes=False),
)
def _embed_norm(
    table_hbm, idx_hbm, out_hbm, idx_v, blk_v, tail_v, sel_i, sel_b, colbuf,
    sem_out, sem_b0, sem_b1,
):
    wid = lax.axis_index("s") * NC + lax.axis_index("c")
    nb = BPW + (wid < NEXTRA).astype(jnp.int32)
    lo = BLK * (BPW * wid + jnp.minimum(wid, NEXTRA))
    hi = jnp.where(wid == NW - 1, jnp.int32(VOCAB), lo + nb * BLK)
    lanes = lax.iota(jnp.int32, LANES)

    def _chunk(ch, cnt):
        pltpu.sync_copy(idx_hbm.at[pl.ds(ch * IDXCH, IDXCH)], idx_v)

        def _sel(v, cnt):
            x = idx_v[pl.ds(v * LANES, LANES)]
            b = lanes + (ch * IDXCH + v * LANES)
            m = (x >= lo) & (x < hi)
            pos = cnt + plsc.cumsum(m.astype(jnp.int32)) - 1
            mm = m & (pos < SELCAP)
            plsc.store_scatter(sel_i, [pos], x, mask=mm)
            plsc.store_scatter(sel_b, [pos], b, mask=mm)
            return cnt + plsc.all_reduce_population_count(m)[0]

        return lax.fori_loop(0, IDXCH // LANES, _sel, cnt)

    nsel = lax.fori_loop(0, BATCH // IDXCH, _chunk, jnp.int32(0))
    ub_sel = (nsel + LANES - 1) // LANES

    def _match(buf, start, width, s_cnt):
        def _scan(u, s_cnt):
            sv = sel_i[pl.ds(u * LANES, LANES)]
            sb = sel_b[pl.ds(u * LANES, LANES)]
            c = sv - start
            valid = (lanes + u * LANES) < nsel
            m = (c >= 0) & (c < width) & valid
            n = plsc.all_reduce_population_count(m)[0]

            def _body(carry):
                mi, sc, n = carry
                m = mi != 0
                l = plsc.all_reduce_ffs(m)[0]
                lsp = jnp.full((LANES,), l, dtype=jnp.int32)
                ci = _take16(c, lsp)[0]
                bb = _take16(sb, lsp)[0]
                civ = jnp.full((LANES,), ci, dtype=jnp.int32)
                vs = [
                    plsc.load_gather(buf, [lanes + LANES * k, civ])
                    for k in range(NV)
                ]
                ssq = vs[0] * vs[0]
                for k in range(1, NV):
                    ssq = ssq + vs[k] * vs[k]
                for sh in (8, 4, 2, 1):
                    ssq = ssq + _take16(ssq, lanes ^ sh)
                y = _rsqrt(ssq + 1e-12)
                for k in range(NV):
                    colbuf[sc, pl.ds(LANES * k, LANES)] = vs[k] * y
                pltpu.async_copy(
                    colbuf.at[sc], out_hbm.at[pl.ds(bb * EMBED, EMBED)], sem_out
                )
                return (jnp.where(lanes == l, jnp.int32(0), mi), sc + 1, n - 1)

            out = lax.while_loop(
                lambda cr: cr[2] > 0, _body, (m.astype(jnp.int32), s_cnt, n)
            )
            return out[1]

        return lax.fori_loop(0, ub_sel, _scan, s_cnt)

    sems = (sem_b0, sem_b1)

    def _start(kk, slot):
        def _f(c):
            pltpu.async_copy(
                table_hbm.at[:, pl.ds(lo + kk * BLK, BLK)],
                blk_v.at[slot],
                sems[slot],
            )
            return c

        lax.cond(kk < nb, _f, lambda c: c, 0)

    def _consume(kk, slot, s_cnt):
        def _f(sc):
            pltpu.make_async_copy(
                table_hbm.at[:, pl.ds(0, BLK)], blk_v.at[slot], sems[slot]
            ).wait()
            return _match(blk_v.at[slot], lo + kk * BLK, jnp.int32(BLK), sc)

        return lax.cond(kk < nb, _f, lambda sc: sc, s_cnt)

    _start(jnp.int32(0), 0)

    def _pair(g, s_cnt):
        _start(2 * g + 1, 1)
        s_cnt = _consume(2 * g, 0, s_cnt)
        _start(2 * g + 2, 0)
        s_cnt = _consume(2 * g + 1, 1, s_cnt)
        return s_cnt

    s_cnt = lax.fori_loop(0, (BPW + 1) // 2 + 1, _pair, jnp.int32(0))

    def _tail(s_cnt):
        pltpu.sync_copy(table_hbm.at[:, pl.ds(TAILS, TAILW)], tail_v)
        return _match(tail_v, jnp.int32(TAILS), jnp.int32(TAILW), s_cnt)

    s_cnt = lax.cond(wid == NW - 1, _tail, lambda sc: sc, s_cnt)

    def _drain(c, carry):
        pltpu.make_async_copy(
            out_hbm.at[pl.ds(0, EMBED)], colbuf.at[0], sem_out
        ).wait()
        return carry

    lax.fori_loop(0, s_cnt, _drain, 0)


def kernel(indices, table):
    idx = indices.astype(jnp.int32)
    res = _embed_norm(table.T, idx)
    return res.reshape(BATCH, EMBED)
